# baseline (speedup 1.0000x reference)
import jax
import jax.numpy as jnp
from jax import lax
from jax.experimental import pallas as pl


_CB = 16


def _pool_body(x_ref, s_ref, o_ref):
    xb = x_ref[0]
    sel = s_ref[...]
    dn = (((2,), (0,)), ((), ()))
    t0 = lax.dot_general(xb, sel[0], dn, preferred_element_type=jnp.float32)
    t1 = lax.dot_general(xb, sel[1], dn, preferred_element_type=jnp.float32)
    t2 = lax.dot_general(xb, sel[2], dn, preferred_element_type=jnp.float32)
    t3 = lax.dot_general(xb, sel[3], dn, preferred_element_type=jnp.float32)
    o_ref[0] = jnp.maximum(jnp.maximum(t0, t1), jnp.maximum(t2, t3))


def kernel(x, indices):
    b, c, s, g = x.shape
    p = indices.shape[1]
    sel = (indices[:, None, :] == jnp.arange(g, dtype=indices.dtype)[None, :, None]
           ).astype(x.dtype)
    return pl.pallas_call(
        _pool_body,
        grid=(b, c // _CB),
        in_specs=[
            pl.BlockSpec((1, _CB, s, g), lambda i, j: (i, j, 0, 0)),
            pl.BlockSpec((4, g, p), lambda i, j: (0, 0, 0)),
        ],
        out_specs=pl.BlockSpec((1, _CB, s, p), lambda i, j: (i, j, 0, 0)),
        out_shape=jax.ShapeDtypeStruct((b, c, s, p), x.dtype),
    )(x, sel)


# speedup vs baseline: 1.1380x; 1.1380x over previous
import jax
import jax.numpy as jnp
from jax import lax
from jax.experimental import pallas as pl


_CB = 64


def _pool_body(x_ref, s_ref, o_ref):
    xb = x_ref[0]
    sel = s_ref[...]
    dn = (((2,), (0,)), ((), ()))
    t0 = lax.dot_general(xb, sel[0], dn, preferred_element_type=jnp.float32)
    t1 = lax.dot_general(xb, sel[1], dn, preferred_element_type=jnp.float32)
    t2 = lax.dot_general(xb, sel[2], dn, preferred_element_type=jnp.float32)
    t3 = lax.dot_general(xb, sel[3], dn, preferred_element_type=jnp.float32)
    o_ref[0] = jnp.maximum(jnp.maximum(t0, t1), jnp.maximum(t2, t3))


def kernel(x, indices):
    b, c, s, g = x.shape
    p = indices.shape[1]
    sel = (indices[:, None, :] == jnp.arange(g, dtype=indices.dtype)[None, :, None]
           ).astype(x.dtype)
    return pl.pallas_call(
        _pool_body,
        grid=(b, c // _CB),
        in_specs=[
            pl.BlockSpec((1, _CB, s, g), lambda i, j: (i, j, 0, 0)),
            pl.BlockSpec((4, g, p), lambda i, j: (0, 0, 0)),
        ],
        out_specs=pl.BlockSpec((1, _CB, s, p), lambda i, j: (i, j, 0, 0)),
        out_shape=jax.ShapeDtypeStruct((b, c, s, p), x.dtype),
    )(x, sel)
